# baseline (speedup 1.0000x reference)
import functools

import jax
import jax.numpy as jnp
from jax import lax
from jax.experimental import pallas as pl
from jax.experimental.pallas import tpu as pltpu
from jax.experimental.pallas import tpu_sc as plsc

B = 1024
N = 1024
TOPO_DIM = 16
BIO_DIM = 112
D = TOPO_DIM + BIO_DIM
K = 32
HIDDEN = 8

NC = 2
NS = 16
NW = NC * NS
BPW = B // NW


_SV_BLK = 8192


def _sv_body(x_ref, gw_ref, o_ref):
    o_ref[...] = jnp.dot(x_ref[...], gw_ref[...],
                         preferred_element_type=jnp.float32)


def _sort_values(topo_flat, gw):
    rows = topo_flat.shape[0]
    return pl.pallas_call(
        _sv_body,
        grid=(rows // _SV_BLK,),
        in_specs=[
            pl.BlockSpec((_SV_BLK, 128), lambda i: (i, 0)),
            pl.BlockSpec((128, 8), lambda i: (0, 0)),
        ],
        out_specs=pl.BlockSpec((_SV_BLK, 8), lambda i: (i, 0)),
        out_shape=jax.ShapeDtypeStruct((rows, 8), jnp.float32),
    )(topo_flat, gw)


def _sort16(k, v):
    return plsc.sort_key_val(k, v, descending=True)


def _merge16(a, b):
    ka, ia = a
    kb, ib = b
    rk = lax.rev(kb, (0,))
    ri = lax.rev(ib, (0,))
    m = ka >= rk
    hk = jnp.where(m, ka, rk)
    hi = jnp.where(m, ia, ri)
    lk = jnp.where(m, rk, ka)
    li = jnp.where(m, ri, ia)
    hk, hi = _sort16(hk, hi)
    lk, li = _sort16(lk, li)
    return hk, hi, lk, li


def _merge32(x, y):
    x0, xi0, x1, xi1 = x
    y0, yi0, y1, yi1 = y
    r0 = lax.rev(y1, (0,))
    ri0 = lax.rev(yi1, (0,))
    r1 = lax.rev(y0, (0,))
    ri1 = lax.rev(yi0, (0,))
    m0 = x0 >= r0
    h0 = jnp.where(m0, x0, r0)
    hi0 = jnp.where(m0, xi0, ri0)
    m1 = x1 >= r1
    h1 = jnp.where(m1, x1, r1)
    hi1 = jnp.where(m1, xi1, ri1)
    m = h0 >= h1
    a = jnp.where(m, h0, h1)
    ai = jnp.where(m, hi0, hi1)
    b = jnp.where(m, h1, h0)
    bi = jnp.where(m, hi1, hi0)
    a, ai = _sort16(a, ai)
    b, bi = _sort16(b, bi)
    return a, ai, b, bi


def _topk_gather(sv, topo2, bio2):
    mesh = plsc.VectorSubcoreMesh(core_axis_name="c", subcore_axis_name="s")

    @functools.partial(
        pl.kernel,
        mesh=mesh,
        compiler_params=pltpu.CompilerParams(
            needs_layout_passes=False, use_tc_tiling_on_sc=False),
        out_type=[
            jax.ShapeDtypeStruct((B * K, TOPO_DIM), jnp.float32),
            jax.ShapeDtypeStruct((B * K, BIO_DIM), jnp.float32),
        ],
        scratch_types=[
            pltpu.VMEM((BPW, N), jnp.float32),
            pltpu.VMEM((K,), jnp.int32),
            pltpu.VMEM((K, TOPO_DIM), jnp.float32),
            pltpu.VMEM((K, BIO_DIM), jnp.float32),
            pltpu.SemaphoreType.DMA,
            pltpu.SemaphoreType.DMA,
        ],
    )
    def kern(sv_hbm, topo_hbm, bio_hbm, xt_hbm, xb_hbm, sv_v, idx_v, trow_v,
             brow_v, sem1, sem2):
        wid = lax.axis_index("s") * NC + lax.axis_index("c")
        base = wid * BPW
        pltpu.sync_copy(sv_hbm.at[pl.ds(base, BPW)], sv_v)
        iota = lax.iota(jnp.int32, 16)

        def row_body(r, carry):
            lists = []
            for c in range(N // 16):
                kv = sv_v[r, pl.ds(c * 16, 16)]
                lists.append(_sort16(kv, iota + (c * 16)))
            l32 = [_merge16(lists[2 * i], lists[2 * i + 1])
                   for i in range(len(lists) // 2)]
            while len(l32) > 1:
                l32 = [_merge32(l32[2 * i], l32[2 * i + 1])
                       for i in range(len(l32) // 2)]
            _, ai, _, bi = l32[0]
            gbase = (base + r) * N
            idx_v[pl.ds(0, 16)] = ai + gbase
            idx_v[pl.ds(16, 16)] = bi + gbase
            cp1 = pltpu.async_copy(bio_hbm.at[idx_v], brow_v, sem1)
            cp2 = pltpu.async_copy(topo_hbm.at[idx_v], trow_v, sem2)
            cp1.wait()
            cp2.wait()
            ob = (base + r) * K
            pltpu.sync_copy(trow_v, xt_hbm.at[pl.ds(ob, K)])
            pltpu.sync_copy(brow_v, xb_hbm.at[pl.ds(ob, K)])
            return carry

        lax.fori_loop(0, BPW, row_body, 0)

    return kern(sv, topo2, bio2)


_HB = 256


def _log_sigmoid(z):
    return jnp.minimum(z, 0.0) - jnp.log(1.0 + jnp.exp(-jnp.abs(z)))


def _head_body(xt_ref, xb_ref, w1t_ref, w1b_ref, b1_ref, w2_ref, b2_ref,
               y_ref, score_ref, loss_ref, acc_ref):
    i = pl.program_id(0)
    nsteps = pl.num_programs(0)
    h = (jnp.dot(xt_ref[...], w1t_ref[...], preferred_element_type=jnp.float32)
         + jnp.dot(xb_ref[...], w1b_ref[...],
                   preferred_element_type=jnp.float32))
    h = jnp.maximum(h + b1_ref[...], 0.0)
    s = jnp.sum(h * w2_ref[...], axis=1) + b2_ref[0, 0]
    score_ref[...] = s[None, None, :]
    yf = y_ref[0, 0, :].astype(jnp.float32)
    s1 = jnp.sum(yf * _log_sigmoid(s))
    s0 = jnp.sum((1.0 - yf) * _log_sigmoid(-s))
    npos = jnp.sum(yf)

    @pl.when(i == 0)
    def _():
        acc_ref[0] = 0.0
        acc_ref[1] = 0.0
        acc_ref[2] = 0.0

    acc_ref[0] += s1
    acc_ref[1] += s0
    acc_ref[2] += npos

    @pl.when(i == nsteps - 1)
    def _():
        tot_pos = acc_ref[2]
        pw = (jnp.float32(B) - tot_pos) / tot_pos
        loss_ref[0, 0] = -(pw * acc_ref[0] + acc_ref[1]) / jnp.float32(B)


def _head(xt2, xb2, w1t, w1b, b1r, w2r, b2r, y2):
    nblk = B // _HB
    score, loss = pl.pallas_call(
        _head_body,
        grid=(nblk,),
        in_specs=[
            pl.BlockSpec((_HB, K * TOPO_DIM), lambda i: (i, 0)),
            pl.BlockSpec((_HB, K * BIO_DIM), lambda i: (i, 0)),
            pl.BlockSpec((K * TOPO_DIM, HIDDEN), lambda i: (0, 0)),
            pl.BlockSpec((K * BIO_DIM, HIDDEN), lambda i: (0, 0)),
            pl.BlockSpec((1, HIDDEN), lambda i: (0, 0)),
            pl.BlockSpec((1, HIDDEN), lambda i: (0, 0)),
            pl.BlockSpec((1, 1), lambda i: (0, 0)),
            pl.BlockSpec((1, 1, _HB), lambda i: (i, 0, 0)),
        ],
        out_specs=[
            pl.BlockSpec((1, 1, _HB), lambda i: (i, 0, 0)),
            pl.BlockSpec(memory_space=pltpu.SMEM),
        ],
        out_shape=[
            jax.ShapeDtypeStruct((nblk, 1, _HB), jnp.float32),
            jax.ShapeDtypeStruct((1, 1), jnp.float32),
        ],
        scratch_shapes=[pltpu.SMEM((3,), jnp.float32)],
    )(xt2, xb2, w1t, w1b, b1r, w2r, b2r, y2)
    return score.reshape(B), loss[0, 0]


def kernel(topo_feat, bio_feat, y, W_t, b_t, W1, b1, W2, b2):
    w = W_t[:, 0]
    lanes = jnp.arange(128)
    gw = jnp.where(lanes[:, None] // 16 == jnp.arange(8)[None, :],
                   jnp.tile(w, 8)[:, None], 0.0).astype(jnp.float32)
    topo_flat = topo_feat.reshape(B * N // 8, 128)
    sv = _sort_values(topo_flat, gw).reshape(B, N)

    topo2 = topo_feat.reshape(B * N, TOPO_DIM)
    bio2 = bio_feat.reshape(B * N, BIO_DIM)
    xt, xb = _topk_gather(sv, topo2, bio2)

    w1s = W1.reshape(K, D, HIDDEN)
    w1t = w1s[:, :TOPO_DIM, :].reshape(K * TOPO_DIM, HIDDEN)
    w1b = w1s[:, TOPO_DIM:, :].reshape(K * BIO_DIM, HIDDEN)
    xt2 = xt.reshape(B, K * TOPO_DIM)
    xb2 = xb.reshape(B, K * BIO_DIM)
    b1r = b1.reshape(1, HIDDEN)
    w2r = W2.reshape(1, HIDDEN)
    b2r = b2.reshape(1, 1)
    y2 = y.reshape(B // _HB, 1, _HB)
    score, loss = _head(xt2, xb2, w1t, w1b, b1r, w2r, b2r, y2)
    return loss, score


# speedup vs baseline: 1.3478x; 1.3478x over previous
import functools

import jax
import jax.numpy as jnp
from jax import lax
from jax.experimental import pallas as pl
from jax.experimental.pallas import tpu as pltpu
from jax.experimental.pallas import tpu_sc as plsc

B = 1024
N = 1024
TOPO_DIM = 16
BIO_DIM = 112
D = TOPO_DIM + BIO_DIM
K = 32
HIDDEN = 8

NC = 2
NS = 16
NW = NC * NS
BPW = B // NW


_NB = 4096


def _sv_body(xf_ref, t_ref, b_ref, gw_ref, sv_ref, ct_ref):
    sv_ref[...] = jnp.dot(xf_ref[...], gw_ref[...],
                          preferred_element_type=jnp.float32)
    ct_ref[:, :TOPO_DIM] = t_ref[...]
    ct_ref[:, TOPO_DIM:] = b_ref[...]


def _sort_values_and_ctab(topo_flat, topo2, bio2, gw):
    rows = topo_flat.shape[0]
    return pl.pallas_call(
        _sv_body,
        grid=(B * N // _NB,),
        in_specs=[
            pl.BlockSpec((_NB // 8, 128), lambda i: (i, 0)),
            pl.BlockSpec((_NB, TOPO_DIM), lambda i: (i, 0)),
            pl.BlockSpec((_NB, BIO_DIM), lambda i: (i, 0)),
            pl.BlockSpec((128, 8), lambda i: (0, 0)),
        ],
        out_specs=[
            pl.BlockSpec((_NB // 8, 8), lambda i: (i, 0)),
            pl.BlockSpec((_NB, D), lambda i: (i, 0)),
        ],
        out_shape=[
            jax.ShapeDtypeStruct((rows, 8), jnp.float32),
            jax.ShapeDtypeStruct((B * N, D), jnp.float32),
        ],
    )(topo_flat, topo2, bio2, gw)


def _sort16(k, v):
    return plsc.sort_key_val(k, v, descending=True)


def _merge16(a, b):
    ka, ia = a
    kb, ib = b
    rk = lax.rev(kb, (0,))
    ri = lax.rev(ib, (0,))
    m = ka >= rk
    hk = jnp.where(m, ka, rk)
    hi = jnp.where(m, ia, ri)
    lk = jnp.where(m, rk, ka)
    li = jnp.where(m, ri, ia)
    hk, hi = _sort16(hk, hi)
    lk, li = _sort16(lk, li)
    return hk, hi, lk, li


def _merge32(x, y):
    x0, xi0, x1, xi1 = x
    y0, yi0, y1, yi1 = y
    r0 = lax.rev(y1, (0,))
    ri0 = lax.rev(yi1, (0,))
    r1 = lax.rev(y0, (0,))
    ri1 = lax.rev(yi0, (0,))
    m0 = x0 >= r0
    h0 = jnp.where(m0, x0, r0)
    hi0 = jnp.where(m0, xi0, ri0)
    m1 = x1 >= r1
    h1 = jnp.where(m1, x1, r1)
    hi1 = jnp.where(m1, xi1, ri1)
    m = h0 >= h1
    a = jnp.where(m, h0, h1)
    ai = jnp.where(m, hi0, hi1)
    b = jnp.where(m, h1, h0)
    bi = jnp.where(m, hi1, hi0)
    a, ai = _sort16(a, ai)
    b, bi = _sort16(b, bi)
    return a, ai, b, bi


def _topk_gather(sv, ctab):
    mesh = plsc.VectorSubcoreMesh(core_axis_name="c", subcore_axis_name="s")

    ngroups = 8
    gr = BPW * K // ngroups
    rpg = BPW // ngroups

    @functools.partial(
        pl.kernel,
        mesh=mesh,
        compiler_params=pltpu.CompilerParams(
            needs_layout_passes=False, use_tc_tiling_on_sc=True),
        out_type=jax.ShapeDtypeStruct((B * K, D), jnp.float32),
        scratch_types=[
            pltpu.VMEM((BPW, N), jnp.float32),
            pltpu.VMEM((ngroups, gr), jnp.int32),
            pltpu.VMEM((2, gr, D), jnp.float32),
            pltpu.SemaphoreType.DMA,
            pltpu.SemaphoreType.DMA,
        ],
    )
    def kern(sv_hbm, ctab_hbm, x_hbm, sv_v, idx_v, row_v, sem0, sem1):
        wid = lax.axis_index("s") * NC + lax.axis_index("c")
        base = wid * BPW
        pltpu.sync_copy(sv_hbm.at[pl.ds(base, BPW)], sv_v)
        iota = lax.iota(jnp.int32, 16)

        def _row_body(g):
            def row_body(r, carry):
                lists = []
                for c in range(N // 16):
                    kv = sv_v[r, pl.ds(c * 16, 16)]
                    lists.append(_sort16(kv, iota + (c * 16)))
                l32 = [_merge16(lists[2 * i], lists[2 * i + 1])
                       for i in range(len(lists) // 2)]
                while len(l32) > 1:
                    l32 = [_merge32(l32[2 * i], l32[2 * i + 1])
                           for i in range(len(l32) // 2)]
                _, ai, _, bi = l32[0]
                gbase = (base + r) * N
                o = (r - g * rpg) * K
                idx_v[g, pl.ds(o, 16)] = ai + gbase
                idx_v[g, pl.ds(o + 16, 16)] = bi + gbase
                return carry
            return row_body

        sems = (sem0, sem1)
        cps = []
        for g in range(ngroups):
            lax.fori_loop(g * rpg, (g + 1) * rpg, _row_body(g), 0)
            s = g % 2
            if g >= 2:
                cps[g - 2].wait()
                ob = base * K + (g - 2) * gr
                pltpu.sync_copy(row_v.at[s], x_hbm.at[pl.ds(ob, gr)])
            ii = idx_v.at[g]
            cps.append(pltpu.async_copy(ctab_hbm.at[ii], row_v.at[s], sems[s]))
        for g in range(ngroups - 2, ngroups):
            s = g % 2
            cps[g].wait()
            ob = base * K + g * gr
            pltpu.sync_copy(row_v.at[s], x_hbm.at[pl.ds(ob, gr)])

    return kern(sv, ctab)


_HB = 256


def _log_sigmoid(z):
    return jnp.minimum(z, 0.0) - jnp.log(1.0 + jnp.exp(-jnp.abs(z)))


def _head_body(x_ref, w1_ref, b1_ref, w2_ref, b2_ref,
               y_ref, score_ref, loss_ref, acc_ref):
    i = pl.program_id(0)
    nsteps = pl.num_programs(0)
    h = jnp.dot(x_ref[...], w1_ref[...], preferred_element_type=jnp.float32)
    h = jnp.maximum(h + b1_ref[...], 0.0)
    s = jnp.sum(h * w2_ref[...], axis=1) + b2_ref[0, 0]
    score_ref[...] = s[None, None, :]
    yf = y_ref[0, 0, :].astype(jnp.float32)
    s1 = jnp.sum(yf * _log_sigmoid(s))
    s0 = jnp.sum((1.0 - yf) * _log_sigmoid(-s))
    npos = jnp.sum(yf)

    @pl.when(i == 0)
    def _():
        acc_ref[0] = 0.0
        acc_ref[1] = 0.0
        acc_ref[2] = 0.0

    acc_ref[0] += s1
    acc_ref[1] += s0
    acc_ref[2] += npos

    @pl.when(i == nsteps - 1)
    def _():
        tot_pos = acc_ref[2]
        pw = (jnp.float32(B) - tot_pos) / tot_pos
        loss_ref[0, 0] = -(pw * acc_ref[0] + acc_ref[1]) / jnp.float32(B)


def _head(x2, w1, b1r, w2r, b2r, y2):
    nblk = B // _HB
    score, loss = pl.pallas_call(
        _head_body,
        grid=(nblk,),
        in_specs=[
            pl.BlockSpec((_HB, K * D), lambda i: (i, 0)),
            pl.BlockSpec((K * D, HIDDEN), lambda i: (0, 0)),
            pl.BlockSpec((1, HIDDEN), lambda i: (0, 0)),
            pl.BlockSpec((1, HIDDEN), lambda i: (0, 0)),
            pl.BlockSpec((1, 1), lambda i: (0, 0)),
            pl.BlockSpec((1, 1, _HB), lambda i: (i, 0, 0)),
        ],
        out_specs=[
            pl.BlockSpec((1, 1, _HB), lambda i: (i, 0, 0)),
            pl.BlockSpec(memory_space=pltpu.SMEM),
        ],
        out_shape=[
            jax.ShapeDtypeStruct((nblk, 1, _HB), jnp.float32),
            jax.ShapeDtypeStruct((1, 1), jnp.float32),
        ],
        scratch_shapes=[pltpu.SMEM((3,), jnp.float32)],
    )(x2, w1, b1r, w2r, b2r, y2)
    return score.reshape(B), loss[0, 0]


def kernel(topo_feat, bio_feat, y, W_t, b_t, W1, b1, W2, b2):
    w = W_t[:, 0]
    lanes = jnp.arange(128)
    gw = jnp.where(lanes[:, None] // 16 == jnp.arange(8)[None, :],
                   jnp.tile(w, 8)[:, None], 0.0).astype(jnp.float32)
    topo_flat = topo_feat.reshape(B * N // 8, 128)
    topo2 = topo_feat.reshape(B * N, TOPO_DIM)
    bio2 = bio_feat.reshape(B * N, BIO_DIM)
    sv8, ctab = _sort_values_and_ctab(topo_flat, topo2, bio2, gw)
    sv = sv8.reshape(B, N)

    x = _topk_gather(sv, ctab)

    x2 = x.reshape(B, K * D)
    b1r = b1.reshape(1, HIDDEN)
    w2r = W2.reshape(1, HIDDEN)
    b2r = b2.reshape(1, 1)
    y2 = y.reshape(B // _HB, 1, _HB)
    score, loss = _head(x2, W1, b1r, w2r, b2r, y2)
    return loss, score
